# baseline (device time: 236036 ns/iter reference)
import jax
import jax.numpy as jnp
from jax import lax
from jax.experimental import pallas as pl
from jax.experimental.pallas import tpu as pltpu

N_DEV = 4
SQ = 2048
SKV = 2048
HALF = SKV // 2
H = 8
DH = 128
D = H * DH
BLK = 64
BPS = SKV // BLK
HBLK = HALF // BLK
KV_TILE = 256
HTILES = HALF // KV_TILE
SCALE = 0.08838834764831843


def _prep_body(x_ref, wq_ref, k_ref, v_ref, q_out, k_out, v_out):
    q = jnp.dot(x_ref[...], wq_ref[...], preferred_element_type=jnp.float32)
    q_out[...] = (q * (SCALE * 1.4426950408889634)).astype(jnp.bfloat16)
    k_out[...] = k_ref[...].astype(jnp.bfloat16)
    v_out[...] = v_ref[...].astype(jnp.bfloat16)


def _attn_body(q_ref, k_hbm, v_hbm, out_ref,
               commr_ref, comml_ref, mask_ref, l_ref,
               local_sems, send_r, recv_r, send_l, recv_l,
               sub_send, sub_recv):
    my = lax.axis_index("i")
    left = lax.rem(my + N_DEV - 1, N_DEV)
    right = lax.rem(my + 1, N_DEV)

    barrier_sem = pltpu.get_barrier_semaphore()
    for nbr in (left, right):
        pl.semaphore_signal(
            barrier_sem, inc=1,
            device_id=(nbr,), device_id_type=pl.DeviceIdType.MESH,
        )
    pl.semaphore_wait(barrier_sem, 2)

    cps = [
        pltpu.make_async_copy(k_hbm.at[0:HALF], commr_ref.at[0, 0],
                              local_sems.at[0]),
        pltpu.make_async_copy(v_hbm.at[0:HALF], commr_ref.at[0, 1],
                              local_sems.at[1]),
        pltpu.make_async_copy(k_hbm.at[HALF:SKV], comml_ref.at[0, 0],
                              local_sems.at[2]),
        pltpu.make_async_copy(v_hbm.at[HALF:SKV], comml_ref.at[0, 1],
                              local_sems.at[3]),
    ]
    for cp in cps:
        cp.start()

    out_ref[...] = jnp.zeros((SQ, D), jnp.float32)
    l_ref[...] = jnp.zeros((SQ, H), jnp.float32)

    row = lax.broadcasted_iota(jnp.int32, (SQ, 1), 0)
    qb = my * BPS + row // BLK
    qm = qb % 3

    def fill_mask(origin, tile_base, blk_off):
        def tile_step(t, _):
            col = lax.broadcasted_iota(jnp.int32, (1, KV_TILE), 1)
            kb = origin * BPS + blk_off + (t * KV_TILE + col) // BLK
            km = kb % 3
            r = qm + km
            keep = (qb == kb) | (kb == 0) | (r == 0) | (r == 3)
            mask_ref[tile_base + t] = keep.astype(jnp.int8)
            return 0
        lax.fori_loop(0, HTILES, tile_step, 0)

    def process(comm_ref, slot, tile_base, t_lo=0, t_hi=HTILES):
        def tile_step(t, _):
            kv_rows = pl.ds(t * KV_TILE, KV_TILE)
            mf = mask_ref[tile_base + t].astype(jnp.float32)
            for h in range(H):
                hd = slice(h * DH, (h + 1) * DH)
                qh = q_ref[:, hd]
                kh = comm_ref[slot, 0, kv_rows, hd]
                vh = comm_ref[slot, 1, kv_rows, hd]
                s = lax.dot_general(
                    qh, kh, (((1,), (1,)), ((), ())),
                    preferred_element_type=jnp.float32,
                )
                p = jnp.exp2(s) * mf
                l_ref[:, h:h + 1] = (
                    l_ref[:, h:h + 1] + jnp.sum(p, axis=1, keepdims=True)
                )
                pv = jnp.dot(
                    p, vh.astype(jnp.float32),
                    preferred_element_type=jnp.float32,
                )
                out_ref[:, hd] = out_ref[:, hd] + pv
            return 0
        lax.fori_loop(t_lo, t_hi, tile_step, 0)

    fill_mask(my, 0, 0)
    fill_mask(my, HTILES, HBLK)
    for cp in cps:
        cp.wait()

    for hop in range(N_DEV - 2):
        s_slot = hop % 2
        r_slot = (hop + 1) % 2
        rdma_r = pltpu.make_async_remote_copy(
            src_ref=commr_ref.at[s_slot],
            dst_ref=commr_ref.at[r_slot],
            send_sem=send_r.at[s_slot],
            recv_sem=recv_r.at[r_slot],
            device_id=(right,),
            device_id_type=pl.DeviceIdType.MESH,
        )
        rdma_l = pltpu.make_async_remote_copy(
            src_ref=comml_ref.at[s_slot],
            dst_ref=comml_ref.at[r_slot],
            send_sem=send_l.at[s_slot],
            recv_sem=recv_l.at[r_slot],
            device_id=(left,),
            device_id_type=pl.DeviceIdType.MESH,
        )
        rdma_r.start()
        rdma_l.start()
        process(commr_ref, s_slot, 0)
        process(comml_ref, s_slot, HTILES)
        fill_mask(lax.rem(my - hop - 1 + N_DEV, N_DEV), 0, 0)
        fill_mask(lax.rem(my + hop + 1, N_DEV), HTILES, HBLK)
        rdma_r.wait()
        rdma_l.wait()

    hop = N_DEV - 2
    s_slot = hop % 2
    r_slot = (hop + 1) % 2
    QROW = HALF // 2
    QT = HTILES // 2
    subs = []
    for i, (comm, ssem, rsem, dev) in enumerate((
        (commr_ref, send_r, recv_r, right),
        (comml_ref, send_l, recv_l, left),
    )):
        r1 = pltpu.make_async_remote_copy(
            src_ref=comm.at[s_slot, :, 0:QROW],
            dst_ref=comm.at[r_slot, :, 0:QROW],
            send_sem=ssem.at[s_slot],
            recv_sem=rsem.at[r_slot],
            device_id=(dev,),
            device_id_type=pl.DeviceIdType.MESH,
        )
        r2 = pltpu.make_async_remote_copy(
            src_ref=comm.at[s_slot, :, QROW:HALF],
            dst_ref=comm.at[r_slot, :, QROW:HALF],
            send_sem=sub_send.at[i],
            recv_sem=sub_recv.at[i],
            device_id=(dev,),
            device_id_type=pl.DeviceIdType.MESH,
        )
        r1.start()
        r2.start()
        subs.append((r1, r2))
    process(commr_ref, s_slot, 0)
    process(comml_ref, s_slot, HTILES)
    fill_mask(lax.rem(my - hop - 1 + N_DEV, N_DEV), 0, 0)
    fill_mask(lax.rem(my + hop + 1, N_DEV), HTILES, HBLK)
    subs[0][0].wait()
    subs[1][0].wait()
    process(commr_ref, r_slot, 0, 0, QT)
    process(comml_ref, r_slot, HTILES, 0, QT)
    subs[0][1].wait()
    subs[1][1].wait()
    process(commr_ref, r_slot, 0, QT, HTILES)
    process(comml_ref, r_slot, HTILES, QT, HTILES)

    for h in range(H):
        hd = slice(h * DH, (h + 1) * DH)
        out_ref[:, hd] = out_ref[:, hd] / l_ref[:, h:h + 1]


def _out_body(ctx_ref, wo_ref, out_ref):
    out_ref[...] = jnp.dot(
        ctx_ref[...], wo_ref[...], preferred_element_type=jnp.float32
    )


def kernel(x, Wq, K_ext, V_ext, Wo):
    x2 = x.reshape(SQ, D)
    k2 = K_ext.reshape(SKV, D)
    v2 = V_ext.reshape(SKV, D)

    qb16, kb16, vb16 = pl.pallas_call(
        _prep_body,
        out_shape=(
            jax.ShapeDtypeStruct((SQ, D), jnp.bfloat16),
            jax.ShapeDtypeStruct((SKV, D), jnp.bfloat16),
            jax.ShapeDtypeStruct((SKV, D), jnp.bfloat16),
        ),
        in_specs=[pl.BlockSpec(memory_space=pltpu.MemorySpace.VMEM)] * 4,
        out_specs=(pl.BlockSpec(memory_space=pltpu.MemorySpace.VMEM),) * 3,
    )(x2, Wq, k2, v2)

    ctx = pl.pallas_call(
        _attn_body,
        out_shape=jax.ShapeDtypeStruct((SQ, D), jnp.float32),
        in_specs=[
            pl.BlockSpec(memory_space=pltpu.MemorySpace.VMEM),
            pl.BlockSpec(memory_space=pltpu.MemorySpace.HBM),
            pl.BlockSpec(memory_space=pltpu.MemorySpace.HBM),
        ],
        out_specs=pl.BlockSpec(memory_space=pltpu.MemorySpace.VMEM),
        scratch_shapes=[
            pltpu.VMEM((2, 2, HALF, D), jnp.bfloat16),
            pltpu.VMEM((2, 2, HALF, D), jnp.bfloat16),
            pltpu.VMEM((2 * HTILES, SQ, KV_TILE), jnp.int8),
            pltpu.VMEM((SQ, H), jnp.float32),
            pltpu.SemaphoreType.DMA((4,)),
            pltpu.SemaphoreType.DMA((2,)),
            pltpu.SemaphoreType.DMA((2,)),
            pltpu.SemaphoreType.DMA((2,)),
            pltpu.SemaphoreType.DMA((2,)),
            pltpu.SemaphoreType.DMA((2,)),
            pltpu.SemaphoreType.DMA((2,)),
        ],
        compiler_params=pltpu.CompilerParams(collective_id=0),
    )(qb16, kb16, vb16)

    out = pl.pallas_call(
        _out_body,
        out_shape=jax.ShapeDtypeStruct((SQ, D), jnp.float32),
        in_specs=[pl.BlockSpec(memory_space=pltpu.MemorySpace.VMEM)] * 2,
        out_specs=pl.BlockSpec(memory_space=pltpu.MemorySpace.VMEM),
    )(ctx, Wo)

    return out.reshape(1, SQ, D)


# device time: 236019 ns/iter; 1.0001x vs baseline; 1.0001x over previous
import os

import jax
import jax.numpy as jnp
from jax import lax
from jax.experimental import pallas as pl
from jax.experimental.pallas import tpu as pltpu

_PROBE_COMM_ONLY = os.environ.get("PROBE") == "comm"

N_DEV = 4
SQ = 2048
SKV = 2048
HALF = SKV // 2
H = 8
DH = 128
D = H * DH
BLK = 64
BPS = SKV // BLK
HBLK = HALF // BLK
KV_TILE = 256
HTILES = HALF // KV_TILE
SCALE = 0.08838834764831843


def _prep_body(x_ref, wq_ref, k_ref, v_ref, q_out, k_out, v_out):
    q = jnp.dot(x_ref[...], wq_ref[...], preferred_element_type=jnp.float32)
    q_out[...] = (q * (SCALE * 1.4426950408889634)).astype(jnp.bfloat16)
    k_out[...] = k_ref[...].astype(jnp.bfloat16)
    v_out[...] = v_ref[...].astype(jnp.bfloat16)


def _attn_body(q_ref, k_hbm, v_hbm, out_ref,
               commr_ref, comml_ref, mask_ref, l_ref,
               local_sems, send_r, recv_r, send_l, recv_l,
               sub_send, sub_recv):
    my = lax.axis_index("i")
    left = lax.rem(my + N_DEV - 1, N_DEV)
    right = lax.rem(my + 1, N_DEV)

    barrier_sem = pltpu.get_barrier_semaphore()
    for nbr in (left, right):
        pl.semaphore_signal(
            barrier_sem, inc=1,
            device_id=(nbr,), device_id_type=pl.DeviceIdType.MESH,
        )
    pl.semaphore_wait(barrier_sem, 2)

    cps = [
        pltpu.make_async_copy(k_hbm.at[0:HALF], commr_ref.at[0, 0],
                              local_sems.at[0]),
        pltpu.make_async_copy(v_hbm.at[0:HALF], commr_ref.at[0, 1],
                              local_sems.at[1]),
        pltpu.make_async_copy(k_hbm.at[HALF:SKV], comml_ref.at[0, 0],
                              local_sems.at[2]),
        pltpu.make_async_copy(v_hbm.at[HALF:SKV], comml_ref.at[0, 1],
                              local_sems.at[3]),
    ]
    for cp in cps:
        cp.start()

    out_ref[...] = jnp.zeros((SQ, D), jnp.float32)
    l_ref[...] = jnp.zeros((SQ, H), jnp.float32)

    row = lax.broadcasted_iota(jnp.int32, (SQ, 1), 0)
    qb = my * BPS + row // BLK
    qm = qb % 3

    def fill_mask(origin, tile_base, blk_off):
        if _PROBE_COMM_ONLY:
            return
        def tile_step(t, _):
            col = lax.broadcasted_iota(jnp.int32, (1, KV_TILE), 1)
            kb = origin * BPS + blk_off + (t * KV_TILE + col) // BLK
            km = kb % 3
            r = qm + km
            keep = (qb == kb) | (kb == 0) | (r == 0) | (r == 3)
            mask_ref[tile_base + t] = keep.astype(jnp.int8)
            return 0
        lax.fori_loop(0, HTILES, tile_step, 0)

    def process(comm_ref, slot, tile_base, t_lo=0, t_hi=HTILES):
        if _PROBE_COMM_ONLY:
            return
        def tile_step(t, _):
            kv_rows = pl.ds(t * KV_TILE, KV_TILE)
            mf = mask_ref[tile_base + t].astype(jnp.float32)
            for h in range(H):
                hd = slice(h * DH, (h + 1) * DH)
                qh = q_ref[:, hd]
                kh = comm_ref[slot, 0, kv_rows, hd]
                vh = comm_ref[slot, 1, kv_rows, hd]
                s = lax.dot_general(
                    qh, kh, (((1,), (1,)), ((), ())),
                    preferred_element_type=jnp.float32,
                )
                p = jnp.exp2(s) * mf
                l_ref[:, h:h + 1] = (
                    l_ref[:, h:h + 1] + jnp.sum(p, axis=1, keepdims=True)
                )
                pv = jnp.dot(
                    p, vh.astype(jnp.float32),
                    preferred_element_type=jnp.float32,
                )
                out_ref[:, hd] = out_ref[:, hd] + pv
            return 0
        lax.fori_loop(t_lo, t_hi, tile_step, 0)

    fill_mask(my, 0, 0)
    fill_mask(my, HTILES, HBLK)
    for cp in cps:
        cp.wait()

    for hop in range(N_DEV - 2):
        s_slot = hop % 2
        r_slot = (hop + 1) % 2
        rdma_r = pltpu.make_async_remote_copy(
            src_ref=commr_ref.at[s_slot],
            dst_ref=commr_ref.at[r_slot],
            send_sem=send_r.at[s_slot],
            recv_sem=recv_r.at[r_slot],
            device_id=(right,),
            device_id_type=pl.DeviceIdType.MESH,
        )
        rdma_l = pltpu.make_async_remote_copy(
            src_ref=comml_ref.at[s_slot],
            dst_ref=comml_ref.at[r_slot],
            send_sem=send_l.at[s_slot],
            recv_sem=recv_l.at[r_slot],
            device_id=(left,),
            device_id_type=pl.DeviceIdType.MESH,
        )
        rdma_r.start()
        rdma_l.start()
        process(commr_ref, s_slot, 0)
        process(comml_ref, s_slot, HTILES)
        fill_mask(lax.rem(my - hop - 1 + N_DEV, N_DEV), 0, 0)
        fill_mask(lax.rem(my + hop + 1, N_DEV), HTILES, HBLK)
        rdma_r.wait()
        rdma_l.wait()

    hop = N_DEV - 2
    s_slot = hop % 2
    r_slot = (hop + 1) % 2
    QROW = HALF // 2
    QT = HTILES // 2
    subs = []
    for i, (comm, ssem, rsem, dev) in enumerate((
        (commr_ref, send_r, recv_r, right),
        (comml_ref, send_l, recv_l, left),
    )):
        r1 = pltpu.make_async_remote_copy(
            src_ref=comm.at[s_slot, :, 0:QROW],
            dst_ref=comm.at[r_slot, :, 0:QROW],
            send_sem=ssem.at[s_slot],
            recv_sem=rsem.at[r_slot],
            device_id=(dev,),
            device_id_type=pl.DeviceIdType.MESH,
        )
        r2 = pltpu.make_async_remote_copy(
            src_ref=comm.at[s_slot, :, QROW:HALF],
            dst_ref=comm.at[r_slot, :, QROW:HALF],
            send_sem=sub_send.at[i],
            recv_sem=sub_recv.at[i],
            device_id=(dev,),
            device_id_type=pl.DeviceIdType.MESH,
        )
        r1.start()
        r2.start()
        subs.append((r1, r2))
    process(commr_ref, s_slot, 0)
    process(comml_ref, s_slot, HTILES)
    fill_mask(lax.rem(my - hop - 1 + N_DEV, N_DEV), 0, 0)
    fill_mask(lax.rem(my + hop + 1, N_DEV), HTILES, HBLK)
    subs[0][0].wait()
    subs[1][0].wait()
    process(commr_ref, r_slot, 0, 0, QT)
    process(comml_ref, r_slot, HTILES, 0, QT)
    subs[0][1].wait()
    subs[1][1].wait()
    process(commr_ref, r_slot, 0, QT, HTILES)
    process(comml_ref, r_slot, HTILES, QT, HTILES)

    for h in range(H):
        hd = slice(h * DH, (h + 1) * DH)
        out_ref[:, hd] = out_ref[:, hd] / l_ref[:, h:h + 1]


def _out_body(ctx_ref, wo_ref, out_ref):
    out_ref[...] = jnp.dot(
        ctx_ref[...], wo_ref[...], preferred_element_type=jnp.float32
    )


def kernel(x, Wq, K_ext, V_ext, Wo):
    x2 = x.reshape(SQ, D)
    k2 = K_ext.reshape(SKV, D)
    v2 = V_ext.reshape(SKV, D)

    qb16, kb16, vb16 = pl.pallas_call(
        _prep_body,
        out_shape=(
            jax.ShapeDtypeStruct((SQ, D), jnp.bfloat16),
            jax.ShapeDtypeStruct((SKV, D), jnp.bfloat16),
            jax.ShapeDtypeStruct((SKV, D), jnp.bfloat16),
        ),
        in_specs=[pl.BlockSpec(memory_space=pltpu.MemorySpace.VMEM)] * 4,
        out_specs=(pl.BlockSpec(memory_space=pltpu.MemorySpace.VMEM),) * 3,
    )(x2, Wq, k2, v2)

    ctx = pl.pallas_call(
        _attn_body,
        out_shape=jax.ShapeDtypeStruct((SQ, D), jnp.float32),
        in_specs=[
            pl.BlockSpec(memory_space=pltpu.MemorySpace.VMEM),
            pl.BlockSpec(memory_space=pltpu.MemorySpace.HBM),
            pl.BlockSpec(memory_space=pltpu.MemorySpace.HBM),
        ],
        out_specs=pl.BlockSpec(memory_space=pltpu.MemorySpace.VMEM),
        scratch_shapes=[
            pltpu.VMEM((2, 2, HALF, D), jnp.bfloat16),
            pltpu.VMEM((2, 2, HALF, D), jnp.bfloat16),
            pltpu.VMEM((2 * HTILES, SQ, KV_TILE), jnp.int8),
            pltpu.VMEM((SQ, H), jnp.float32),
            pltpu.SemaphoreType.DMA((4,)),
            pltpu.SemaphoreType.DMA((2,)),
            pltpu.SemaphoreType.DMA((2,)),
            pltpu.SemaphoreType.DMA((2,)),
            pltpu.SemaphoreType.DMA((2,)),
            pltpu.SemaphoreType.DMA((2,)),
            pltpu.SemaphoreType.DMA((2,)),
        ],
        compiler_params=pltpu.CompilerParams(collective_id=0),
    )(qb16, kb16, vb16)

    out = pl.pallas_call(
        _out_body,
        out_shape=jax.ShapeDtypeStruct((SQ, D), jnp.float32),
        in_specs=[pl.BlockSpec(memory_space=pltpu.MemorySpace.VMEM)] * 2,
        out_specs=pl.BlockSpec(memory_space=pltpu.MemorySpace.VMEM),
    )(ctx, Wo)

    return out.reshape(1, SQ, D)
